# Initial kernel scaffold; baseline (speedup 1.0000x reference)
import jax
import jax.numpy as jnp
from jax.experimental import pallas as pl


def kernel(type_id, embed_table, type_chart, chart_W, chart_b, combine_W, combine_b):
    raise NotImplementedError("write your pallas kernel here")


# speedup vs baseline: 4.5861x; 4.5861x over previous
import functools

import jax
import jax.numpy as jnp
from jax import lax
from jax.experimental import pallas as pl
from jax.experimental.pallas import tpu as pltpu
from jax.experimental.pallas import tpu_sc as plsc

_NUM_TYPES = 1000
_EMBED_DIM = 128
_HALF_DIM = _EMBED_DIM // 2
_BATCH = 16384

_NC = 2
_NS = 16
_NW = _NC * _NS
_B_PER_W = _BATCH // _NW
_CHUNK = 128
_NCHUNK = _B_PER_W // _CHUNK


def _fuse_table_body(embed_ref, chart_ref, cw_ref, cb_ref, w1_ref, w2_ref,
                     comb_b_ref, out_ref):
    p = lax.dot_general(chart_ref[...], cw_ref[...], (((1,), (1,)), ((), ())),
                        preferred_element_type=jnp.float32)
    chart_part = lax.dot_general(p, w2_ref[...], (((1,), (1,)), ((), ())),
                                 preferred_element_type=jnp.float32)
    base_part = lax.dot_general(embed_ref[...], w1_ref[...],
                                (((1,), (1,)), ((), ())),
                                preferred_element_type=jnp.float32)
    bias = lax.dot_general(cb_ref[...], w2_ref[...], (((1,), (1,)), ((), ())),
                           preferred_element_type=jnp.float32) + comb_b_ref[...]
    out_ref[...] = base_part + chart_part + bias


def _gather_body(table_hbm, idx_hbm, out_hbm, idx_v, rows_v, sem):
    wid = lax.axis_index("s") * _NC + lax.axis_index("c")
    base = wid * _B_PER_W
    pltpu.sync_copy(idx_hbm.at[pl.ds(wid * _NCHUNK, _NCHUNK)], idx_v)
    copies = [
        pltpu.async_copy(table_hbm.at[idx_v.at[j]],
                         rows_v.at[pl.ds(j * _CHUNK, _CHUNK)], sem)
        for j in range(_NCHUNK)
    ]
    for c in copies:
        c.wait()
    pltpu.sync_copy(rows_v, out_hbm.at[pl.ds(base, _B_PER_W)])


def kernel(type_id, embed_table, type_chart, chart_W, chart_b, combine_W,
           combine_b):
    w1 = combine_W[:, :_EMBED_DIM]
    w2 = combine_W[:, _EMBED_DIM:]
    fused = pl.pallas_call(
        _fuse_table_body,
        out_shape=jax.ShapeDtypeStruct((_NUM_TYPES, _EMBED_DIM), jnp.float32),
    )(embed_table, type_chart, chart_W, chart_b.reshape(1, _HALF_DIM), w1, w2,
      combine_b.reshape(1, _EMBED_DIM))

    idx = type_id.astype(jnp.int32).reshape(_NW * _NCHUNK, _CHUNK)

    mesh = plsc.VectorSubcoreMesh(core_axis_name="c", subcore_axis_name="s")
    gather = pl.kernel(
        _gather_body,
        out_type=jax.ShapeDtypeStruct((_BATCH, _EMBED_DIM), jnp.float32),
        mesh=mesh,
        scratch_types=[
            pltpu.VMEM((_NCHUNK, _CHUNK), jnp.int32),
            pltpu.VMEM((_B_PER_W, _EMBED_DIM), jnp.float32),
            pltpu.SemaphoreType.DMA,
        ],
    )
    return gather(fused, idx)
